# Initial kernel scaffold; baseline (speedup 1.0000x reference)
import jax
import jax.numpy as jnp
from jax.experimental import pallas as pl


def kernel(x, edge_index, W, a_src, a_dst, lin_W, lin_b):
    raise NotImplementedError("write your pallas kernel here")


# speedup vs baseline: 4.5576x; 4.5576x over previous
import functools

import jax
import jax.numpy as jnp
from jax import lax
from jax.experimental import pallas as pl
from jax.experimental.pallas import tpu as pltpu
from jax.experimental.pallas import tpu_sc as plsc

_N_BLK = 1000
_NPAD = 10240
_DUMP = 10008
_EC = 20480


def _prep_body(x_ref, w_ref, a8_ref, h_ref, al_ref):
    K = w_ref.shape[0]
    OUT = w_ref.shape[2]
    al = None
    for k in range(K):
        hk = jnp.dot(x_ref[...], w_ref[k], preferred_element_type=jnp.float32)
        h_ref[k] = hk
        con = jnp.dot(
            hk, a8_ref[...][k * OUT : (k + 1) * OUT, :],
            preferred_element_type=jnp.float32,
        )
        al = con if al is None else al + con
    al_ref[...] = al


def _prep(x, W, A8):
    N, C = x.shape
    K = W.shape[0]
    OUT = W.shape[2]
    K2 = A8.shape[1]
    return pl.pallas_call(
        _prep_body,
        grid=(N // _N_BLK,),
        in_specs=[
            pl.BlockSpec((_N_BLK, C), lambda i: (i, 0)),
            pl.BlockSpec((K, C, OUT), lambda i: (0, 0, 0)),
            pl.BlockSpec((K * OUT, K2), lambda i: (0, 0)),
        ],
        out_specs=[
            pl.BlockSpec((K, _N_BLK, OUT), lambda i: (0, i, 0)),
            pl.BlockSpec((_N_BLK, K2), lambda i: (i, 0)),
        ],
        out_shape=[
            jax.ShapeDtypeStruct((K, N, OUT), jnp.float32),
            jax.ShapeDtypeStruct((N, K2), jnp.float32),
        ],
    )(x, W, A8)


def _sc_body(eip2, haug, att_o, feat_o,
             sbuf, dbuf, ebuf, abuf, ts2d, td2d, dnv, zbuf, rows0,
             sem_g0, den_sp, feat_sp):
    s = lax.axis_index("s")
    z16 = jnp.zeros((16,), jnp.float32)
    i16 = lax.iota(jnp.int32, 16)

    def _zb(r, _):
        for l in range(8):
            zbuf[r, pl.ds(l * 16, 16)] = z16
        return 0

    lax.fori_loop(0, 4, _zb, 0)

    def _gather_strip(g, k):
        idx = i16 + (s * 160 + g * 16)
        pltpu.async_copy(eip2.at[idx], sbuf, sem_g0).wait()
        idx = i16 + (2560 + s * 160 + g * 16)
        pltpu.async_copy(eip2.at[idx], dbuf, sem_g0).wait()

        def _erow(r, _):
            for l in range(8):
                sl = pl.ds(l * 16, 16)
                si = sbuf[r, sl]
                di = dbuf[r, sl]
                gs = plsc.load_gather(
                    ts2d,
                    [lax.shift_right_logical(si, 7), jnp.bitwise_and(si, 127)],
                )
                gd = plsc.load_gather(
                    td2d,
                    [lax.shift_right_logical(di, 7), jnp.bitwise_and(di, 127)],
                )
                lo = gs + gd
                lo = jnp.where(lo >= 0, lo, lo * jnp.float32(0.2))
                ebuf[r, sl] = jnp.exp(lo)
            return 0

        lax.fori_loop(0, 16, _erow, 0)

    for k in range(4):
        def _ag_s(g, _):
            idx = i16 + (10000 + g * 16)
            pltpu.async_copy(
                haug.at[k].at[idx], ts2d.at[pl.ds(g * 16, 16)], sem_g0
            ).wait()
            return 0

        lax.fori_loop(0, 5, _ag_s, 0)

        def _ag_d(g, _):
            idx = i16 + (10080 + g * 16)
            pltpu.async_copy(
                haug.at[k].at[idx], td2d.at[pl.ds(g * 16, 16)], sem_g0
            ).wait()
            return 0

        lax.fori_loop(0, 5, _ag_d, 0)

        for j in range(5):
            pltpu.sync_copy(zbuf.at[0], den_sp.at[pl.ds(s * 640 + j * 128, 128)])

        def _zf(j, _):
            pltpu.sync_copy(zbuf, feat_sp.at[pl.ds(s * 640 + j * 4, 4)])
            return 0

        lax.fori_loop(0, 160, _zf, 0)
        plsc.subcore_barrier()

        def _p1(g, _):
            _gather_strip(g, k)

            def _drow(r, _):
                pltpu.sync_copy(ebuf.at[r], den_sp.at[dbuf.at[r]], add=True)
                return 0

            lax.fori_loop(0, 16, _drow, 0)
            return 0

        lax.fori_loop(0, 10, _p1, 0)
        plsc.subcore_barrier()

        def _dcp(j, _):
            pltpu.sync_copy(den_sp.at[pl.ds(j * 128, 128)], dnv.at[j])
            return 0

        lax.fori_loop(0, 80, _dcp, 0)

        def _p2(g, _):
            _gather_strip(g, k)

            def _arow(r, _):
                for l in range(8):
                    sl = pl.ds(l * 16, 16)
                    di = dbuf[r, sl]
                    dv = plsc.load_gather(
                        dnv,
                        [lax.shift_right_logical(di, 7), jnp.bitwise_and(di, 127)],
                    )
                    abuf[r, sl] = ebuf[r, sl] / (dv + jnp.float32(1e-16))
                return 0

            lax.fori_loop(0, 16, _arow, 0)
            pltpu.sync_copy(abuf, att_o.at[k, s, pl.ds(g * 16, 16)])

            def _feat(rs, _):
                r = lax.shift_right_logical(rs, 3)
                q = jnp.bitwise_and(rs, 7)
                sl = pl.ds(q * 16, 16)
                si = sbuf[r, sl]
                pltpu.async_copy(haug.at[k].at[si], rows0, sem_g0).wait()
                ev16 = ebuf[r, sl]
                for lane in range(16):
                    ev = ev16[lane]
                    for l in range(8):
                        cs = pl.ds(l * 16, 16)
                        rows0[lane, cs] = rows0[lane, cs] * ev
                di = dbuf[r, sl]
                pltpu.sync_copy(rows0, feat_sp.at[di], add=True)
                return 0

            lax.fori_loop(0, 128, _feat, 0)
            return 0

        lax.fori_loop(0, 10, _p2, 0)
        plsc.subcore_barrier()

        def _dump(r2, _):
            base = s * 640 + r2 * 16
            for j in range(4):
                pltpu.sync_copy(
                    feat_sp.at[pl.ds(base + j * 4, 4)], rows0.at[pl.ds(j * 4, 4)]
                )
            brow = lax.shift_right_logical(base, 7)
            bcol = jnp.bitwise_and(base, 127)
            dv16 = dnv[brow, pl.ds(bcol, 16)]
            iv16 = jnp.float32(1.0) / (dv16 + jnp.float32(1e-16))
            for lane in range(16):
                iv = iv16[lane]
                for l in range(8):
                    cs = pl.ds(l * 16, 16)
                    rows0[lane, cs] = rows0[lane, cs] * iv
            pltpu.sync_copy(rows0, feat_o.at[k, pl.ds(base, 16)])
            return 0

        lax.fori_loop(0, 40, _dump, 0)


def _sc_edges(eip2, haug):
    K = haug.shape[0]
    mesh = plsc.VectorSubcoreMesh(
        core_axis_name="c", subcore_axis_name="s", num_cores=1, num_subcores=16
    )
    f = pl.kernel(
        _sc_body,
        compiler_params=pltpu.CompilerParams(needs_layout_passes=False),
        out_type=[
            jax.ShapeDtypeStruct((K, 16, 160, 128), jnp.float32),
            jax.ShapeDtypeStruct((K, _NPAD, 128), jnp.float32),
        ],
        mesh=mesh,
        scratch_types=[
            pltpu.VMEM((16, 128), jnp.int32),
            pltpu.VMEM((16, 128), jnp.int32),
            pltpu.VMEM((16, 128), jnp.float32),
            pltpu.VMEM((16, 128), jnp.float32),
            pltpu.VMEM((80, 128), jnp.float32),
            pltpu.VMEM((80, 128), jnp.float32),
            pltpu.VMEM((80, 128), jnp.float32),
            pltpu.VMEM((4, 128), jnp.float32),
            pltpu.VMEM((16, 128), jnp.float32),
            pltpu.SemaphoreType.DMA,
            pltpu.VMEM_SHARED((_NPAD,), jnp.float32),
            pltpu.VMEM_SHARED((_NPAD, 128), jnp.float32),
        ],
    )
    return f(eip2, haug)


def _combine_body(feat_ref, wt_ref, b_ref, out_ref):
    K = feat_ref.shape[0]
    acc = None
    for k in range(K):
        f = feat_ref[k]
        fl = jnp.where(f >= 0, f, 0.01 * f)
        acc = fl if acc is None else acc + fl
    acc = acc * (1.0 / K)
    out_ref[...] = (
        jnp.dot(acc, wt_ref[...], preferred_element_type=jnp.float32) + b_ref[...]
    )


def _combine(feat, lin_Wt, lin_b2, N):
    K, NP, OUT = feat.shape
    return pl.pallas_call(
        _combine_body,
        grid=(N // _N_BLK,),
        in_specs=[
            pl.BlockSpec((K, _N_BLK, OUT), lambda i: (0, i, 0)),
            pl.BlockSpec((OUT, OUT), lambda i: (0, 0)),
            pl.BlockSpec((1, OUT), lambda i: (0, 0)),
        ],
        out_specs=pl.BlockSpec((_N_BLK, OUT), lambda i: (i, 0)),
        out_shape=jax.ShapeDtypeStruct((N, OUT), jnp.float32),
    )(feat, lin_Wt, lin_b2)


def _attmean_body(a_ref, out_ref):
    K = a_ref.shape[0]
    acc = None
    for k in range(K):
        ak = a_ref[k]
        acc = ak if acc is None else acc + ak
    out_ref[...] = acc * (1.0 / K)


def _attmean(att4):
    K, NB, B = att4.shape
    return pl.pallas_call(
        _attmean_body,
        grid=(NB // 8,),
        in_specs=[pl.BlockSpec((K, 8, B), lambda i: (0, i, 0))],
        out_specs=pl.BlockSpec((8, B), lambda i: (i, 0)),
        out_shape=jax.ShapeDtypeStruct((NB, B), jnp.float32),
    )(att4)


def kernel(x, edge_index, W, a_src, a_dst, lin_W, lin_b):
    N, C = x.shape
    K, _, OUT = W.shape
    E = edge_index.shape[1]

    As = jax.scipy.linalg.block_diag(*[a_src[k][:, None] for k in range(K)])
    Ad = jax.scipy.linalg.block_diag(*[a_dst[k][:, None] for k in range(K)])
    A8 = jnp.concatenate([As, Ad], axis=1)

    h, al = _prep(x, W, A8)

    alT = jnp.pad(al.T, ((0, 0), (0, _NPAD - N)))
    alS = alT[:K].reshape(K, _NPAD // 128, 128)
    alD = alT[K:].reshape(K, _NPAD // 128, 128)
    haug = jnp.concatenate([h, alS, alD], axis=1)

    ec = E // 16
    src = edge_index[0].reshape(16, ec)
    dst = edge_index[1].reshape(16, ec)
    srcp = jnp.pad(src, ((0, 0), (0, _EC - ec)))
    dstp = jnp.pad(dst, ((0, 0), (0, _EC - ec)), constant_values=_DUMP)
    eip2 = jnp.concatenate([srcp, dstp]).reshape(2 * 16 * 160, 128)

    att_o, feat = _sc_edges(eip2, haug)

    attm = _attmean(att_o.reshape(K, 320, 1024))
    att_ave = attm.reshape(16, _EC)[:, :ec].reshape(E)
    x_ave = _combine(feat, lin_Wt=lin_W.T, lin_b2=lin_b.reshape(1, OUT), N=N)
    return (x_ave, att_ave)


# speedup vs baseline: 6.0142x; 1.3196x over previous
import functools

import jax
import jax.numpy as jnp
from jax import lax
from jax.experimental import pallas as pl
from jax.experimental.pallas import tpu as pltpu
from jax.experimental.pallas import tpu_sc as plsc

_N_BLK = 1000
_NPAD = 10240
_DUMP = 10008
_EC = 20480


def _prep_body(x_ref, w_ref, a8_ref, h_ref, al_ref):
    K = w_ref.shape[0]
    OUT = w_ref.shape[2]
    al = None
    for k in range(K):
        hk = jnp.dot(x_ref[...], w_ref[k], preferred_element_type=jnp.float32)
        h_ref[k] = hk
        con = jnp.dot(
            hk, a8_ref[...][k * OUT : (k + 1) * OUT, :],
            preferred_element_type=jnp.float32,
        )
        al = con if al is None else al + con
    al_ref[...] = al


def _prep(x, W, A8):
    N, C = x.shape
    K = W.shape[0]
    OUT = W.shape[2]
    K2 = A8.shape[1]
    return pl.pallas_call(
        _prep_body,
        grid=(N // _N_BLK,),
        in_specs=[
            pl.BlockSpec((_N_BLK, C), lambda i: (i, 0)),
            pl.BlockSpec((K, C, OUT), lambda i: (0, 0, 0)),
            pl.BlockSpec((K * OUT, K2), lambda i: (0, 0)),
        ],
        out_specs=[
            pl.BlockSpec((K, _N_BLK, OUT), lambda i: (0, i, 0)),
            pl.BlockSpec((_N_BLK, K2), lambda i: (i, 0)),
        ],
        out_shape=[
            jax.ShapeDtypeStruct((K, N, OUT), jnp.float32),
            jax.ShapeDtypeStruct((N, K2), jnp.float32),
        ],
    )(x, W, A8)


def _sc_body(eip2, haug, att_o, feat_o,
             sbuf, dbuf, ebuf, abuf, ts2d, td2d, dnv, zbuf, rows0, rows1,
             sem_g0, sem_g1, den_sp, feat_sp):
    s = lax.axis_index("s")
    z16 = jnp.zeros((16,), jnp.float32)
    i16 = lax.iota(jnp.int32, 16)

    def _zb(r, _):
        for l in range(8):
            zbuf[r, pl.ds(l * 16, 16)] = z16
        return 0

    lax.fori_loop(0, 4, _zb, 0)

    def _gather_strip(g, k):
        idx = i16 + (s * 160 + g * 16)
        pltpu.async_copy(eip2.at[idx], sbuf, sem_g0).wait()
        idx = i16 + (2560 + s * 160 + g * 16)
        pltpu.async_copy(eip2.at[idx], dbuf, sem_g0).wait()

        def _erow(r, _):
            for l in range(8):
                sl = pl.ds(l * 16, 16)
                si = sbuf[r, sl]
                di = dbuf[r, sl]
                gs = plsc.load_gather(
                    ts2d,
                    [lax.shift_right_logical(si, 7), jnp.bitwise_and(si, 127)],
                )
                gd = plsc.load_gather(
                    td2d,
                    [lax.shift_right_logical(di, 7), jnp.bitwise_and(di, 127)],
                )
                lo = gs + gd
                lo = jnp.where(lo >= 0, lo, lo * jnp.float32(0.2))
                ebuf[r, sl] = jnp.exp(lo)
            return 0

        lax.fori_loop(0, 16, _erow, 0)

    for k in range(4):
        def _ag_s(g, _):
            idx = i16 + (10000 + g * 16)
            pltpu.async_copy(
                haug.at[k].at[idx], ts2d.at[pl.ds(g * 16, 16)], sem_g0
            ).wait()
            return 0

        lax.fori_loop(0, 5, _ag_s, 0)

        def _ag_d(g, _):
            idx = i16 + (10080 + g * 16)
            pltpu.async_copy(
                haug.at[k].at[idx], td2d.at[pl.ds(g * 16, 16)], sem_g0
            ).wait()
            return 0

        lax.fori_loop(0, 5, _ag_d, 0)

        for j in range(5):
            pltpu.sync_copy(zbuf.at[0], den_sp.at[pl.ds(s * 640 + j * 128, 128)])

        def _zf(j, _):
            pltpu.sync_copy(zbuf, feat_sp.at[pl.ds(s * 640 + j * 4, 4)])
            return 0

        lax.fori_loop(0, 160, _zf, 0)
        plsc.subcore_barrier()

        def _p1(g, _):
            _gather_strip(g, k)

            def _drow(r, _):
                pltpu.sync_copy(ebuf.at[r], den_sp.at[dbuf.at[r]], add=True)
                return 0

            lax.fori_loop(0, 16, _drow, 0)
            return 0

        lax.fori_loop(0, 10, _p1, 0)
        plsc.subcore_barrier()

        def _dcp(j, _):
            pltpu.sync_copy(den_sp.at[pl.ds(j * 128, 128)], dnv.at[j])
            return 0

        lax.fori_loop(0, 80, _dcp, 0)

        def _p2(g, _):
            _gather_strip(g, k)

            def _arow(r, _):
                for l in range(8):
                    sl = pl.ds(l * 16, 16)
                    di = dbuf[r, sl]
                    dv = plsc.load_gather(
                        dnv,
                        [lax.shift_right_logical(di, 7), jnp.bitwise_and(di, 127)],
                    )
                    abuf[r, sl] = ebuf[r, sl] / (dv + jnp.float32(1e-16))
                return 0

            lax.fori_loop(0, 16, _arow, 0)
            pltpu.sync_copy(abuf, att_o.at[k, s, pl.ds(g * 16, 16)])

            def _scale_out(buf, r, sl):
                ev16 = ebuf[r, sl]
                for lane in range(16):
                    ev = ev16[lane]
                    for l in range(8):
                        cs = pl.ds(l * 16, 16)
                        buf[lane, cs] = buf[lane, cs] * ev
                di = dbuf[r, sl]
                pltpu.sync_copy(buf, feat_sp.at[di], add=True)

            def _feat(rp, _):
                rs0 = rp * 2
                r0 = lax.shift_right_logical(rs0, 3)
                q0 = jnp.bitwise_and(rs0, 7)
                sl0 = pl.ds(q0 * 16, 16)
                r1 = lax.shift_right_logical(rs0 + 1, 3)
                q1 = jnp.bitwise_and(rs0 + 1, 7)
                sl1 = pl.ds(q1 * 16, 16)
                cp0 = pltpu.async_copy(haug.at[k].at[sbuf[r0, sl0]], rows0, sem_g0)
                cp1 = pltpu.async_copy(haug.at[k].at[sbuf[r1, sl1]], rows1, sem_g1)
                cp0.wait()
                _scale_out(rows0, r0, sl0)
                cp1.wait()
                _scale_out(rows1, r1, sl1)
                return 0

            lax.fori_loop(0, 64, _feat, 0)
            return 0

        lax.fori_loop(0, 10, _p2, 0)
        plsc.subcore_barrier()

        def _dump(r2, _):
            base = s * 640 + r2 * 16
            for j in range(4):
                pltpu.sync_copy(
                    feat_sp.at[pl.ds(base + j * 4, 4)], rows0.at[pl.ds(j * 4, 4)]
                )
            brow = lax.shift_right_logical(base, 7)
            bcol = jnp.bitwise_and(base, 127)
            dv16 = dnv[brow, pl.ds(bcol, 16)]
            iv16 = jnp.float32(1.0) / (dv16 + jnp.float32(1e-16))
            for lane in range(16):
                iv = iv16[lane]
                for l in range(8):
                    cs = pl.ds(l * 16, 16)
                    rows0[lane, cs] = rows0[lane, cs] * iv
            pltpu.sync_copy(rows0, feat_o.at[k, pl.ds(base, 16)])
            return 0

        lax.fori_loop(0, 40, _dump, 0)


def _sc_edges(eip2, haug):
    K = haug.shape[0]
    mesh = plsc.VectorSubcoreMesh(
        core_axis_name="c", subcore_axis_name="s", num_cores=1, num_subcores=16
    )
    f = pl.kernel(
        _sc_body,
        compiler_params=pltpu.CompilerParams(needs_layout_passes=False),
        out_type=[
            jax.ShapeDtypeStruct((K, 16, 160, 128), jnp.float32),
            jax.ShapeDtypeStruct((K, _NPAD, 128), jnp.float32),
        ],
        mesh=mesh,
        scratch_types=[
            pltpu.VMEM((16, 128), jnp.int32),
            pltpu.VMEM((16, 128), jnp.int32),
            pltpu.VMEM((16, 128), jnp.float32),
            pltpu.VMEM((16, 128), jnp.float32),
            pltpu.VMEM((80, 128), jnp.float32),
            pltpu.VMEM((80, 128), jnp.float32),
            pltpu.VMEM((80, 128), jnp.float32),
            pltpu.VMEM((4, 128), jnp.float32),
            pltpu.VMEM((16, 128), jnp.float32),
            pltpu.VMEM((16, 128), jnp.float32),
            pltpu.SemaphoreType.DMA,
            pltpu.SemaphoreType.DMA,
            pltpu.VMEM_SHARED((_NPAD,), jnp.float32),
            pltpu.VMEM_SHARED((_NPAD, 128), jnp.float32),
        ],
    )
    return f(eip2, haug)


def _combine_body(feat_ref, wt_ref, b_ref, out_ref):
    K = feat_ref.shape[0]
    acc = None
    for k in range(K):
        f = feat_ref[k]
        fl = jnp.where(f >= 0, f, 0.01 * f)
        acc = fl if acc is None else acc + fl
    acc = acc * (1.0 / K)
    out_ref[...] = (
        jnp.dot(acc, wt_ref[...], preferred_element_type=jnp.float32) + b_ref[...]
    )


def _combine(feat, lin_Wt, lin_b2, N):
    K, NP, OUT = feat.shape
    return pl.pallas_call(
        _combine_body,
        grid=(N // _N_BLK,),
        in_specs=[
            pl.BlockSpec((K, _N_BLK, OUT), lambda i: (0, i, 0)),
            pl.BlockSpec((OUT, OUT), lambda i: (0, 0)),
            pl.BlockSpec((1, OUT), lambda i: (0, 0)),
        ],
        out_specs=pl.BlockSpec((_N_BLK, OUT), lambda i: (i, 0)),
        out_shape=jax.ShapeDtypeStruct((N, OUT), jnp.float32),
    )(feat, lin_Wt, lin_b2)


def _attmean_body(a_ref, out_ref):
    K = a_ref.shape[0]
    acc = None
    for k in range(K):
        ak = a_ref[k]
        acc = ak if acc is None else acc + ak
    out_ref[...] = acc * (1.0 / K)


def _attmean(att4):
    K, NB, B = att4.shape
    return pl.pallas_call(
        _attmean_body,
        grid=(NB // 8,),
        in_specs=[pl.BlockSpec((K, 8, B), lambda i: (0, i, 0))],
        out_specs=pl.BlockSpec((8, B), lambda i: (i, 0)),
        out_shape=jax.ShapeDtypeStruct((NB, B), jnp.float32),
    )(att4)


def kernel(x, edge_index, W, a_src, a_dst, lin_W, lin_b):
    N, C = x.shape
    K, _, OUT = W.shape
    E = edge_index.shape[1]

    As = jax.scipy.linalg.block_diag(*[a_src[k][:, None] for k in range(K)])
    Ad = jax.scipy.linalg.block_diag(*[a_dst[k][:, None] for k in range(K)])
    A8 = jnp.concatenate([As, Ad], axis=1)

    h, al = _prep(x, W, A8)

    alT = jnp.pad(al.T, ((0, 0), (0, _NPAD - N)))
    alS = alT[:K].reshape(K, _NPAD // 128, 128)
    alD = alT[K:].reshape(K, _NPAD // 128, 128)
    haug = jnp.concatenate([h, alS, alD], axis=1)

    ec = E // 16
    src = edge_index[0].reshape(16, ec)
    dst = edge_index[1].reshape(16, ec)
    srcp = jnp.pad(src, ((0, 0), (0, _EC - ec)))
    dstp = jnp.pad(dst, ((0, 0), (0, _EC - ec)), constant_values=_DUMP)
    eip2 = jnp.concatenate([srcp, dstp]).reshape(2 * 16 * 160, 128)

    att_o, feat = _sc_edges(eip2, haug)

    attm = _attmean(att_o.reshape(K, 320, 1024))
    att_ave = attm.reshape(16, _EC)[:, :ec].reshape(E)
    x_ave = _combine(feat, lin_Wt=lin_W.T, lin_b2=lin_b.reshape(1, OUT), N=N)
    return (x_ave, att_ave)
